# baseline (speedup 1.0000x reference)
import functools

import jax
import jax.numpy as jnp
from jax import lax
from jax.experimental import pallas as pl
from jax.experimental.pallas import tpu as pltpu
from jax.experimental.pallas import tpu_sc as plsc

B, Y, Z, X = 64, 32768, 1000, 256
YB = 4096
NB = Y // YB

ZP = 1024


def _argmax_body(x_ref, w_ref, age_ref, idx_ref, nrm_ref, bv_ref):
    i = pl.program_id(0)

    x = x_ref[...]
    xn = x / jnp.maximum(jnp.sqrt(jnp.sum(x * x, axis=1, keepdims=True)), 1e-12)

    w = w_ref[...]
    r = jnp.maximum(jnp.sqrt(jnp.sum(w * w, axis=1, keepdims=True)), 1e-12)
    wn = w / r

    y = lax.dot_general(xn, wn, (((1,), (1,)), ((), ())),
                        preferred_element_type=jnp.float32)
    age = age_ref[...]
    y = y * jnp.where(age >= 1.0, 1.0, 0.0)

    m = jnp.max(y, axis=1, keepdims=True)
    match = (y == m).astype(jnp.float32)
    colsf = lax.broadcasted_iota(jnp.int32, (YB, 1), 0).astype(jnp.float32)
    rhs = jnp.concatenate([colsf, r, jnp.ones((YB, 1), jnp.float32)], axis=1)
    fast = lax.dot_general(match, rhs, (((1,), (0,)), ((), ())),
                           precision=lax.Precision.HIGHEST,
                           preferred_element_type=jnp.float32)
    cnt = fast[:, 2:3]

    def _unique_max():
        return fast[:, 0:1].astype(jnp.int32), fast[:, 1:2]

    def _tie_break():
        cols = lax.broadcasted_iota(jnp.int32, (B, YB), 1)
        a_s = jnp.min(jnp.where(y == m, cols, Y), axis=1, keepdims=True)
        onehot = (cols == a_s).astype(jnp.float32)
        wr_s = lax.dot_general(onehot, r, (((1,), (0,)), ((), ())),
                               precision=lax.Precision.HIGHEST,
                               preferred_element_type=jnp.float32)
        return a_s, wr_s

    a, wr = lax.cond(jnp.all(cnt < 1.5), _unique_max, _tie_break)

    @pl.when(i == 0)
    def _():
        bv_ref[...] = jnp.full((B, 1), -jnp.inf, jnp.float32)

    better = m > bv_ref[...]
    bv_ref[...] = jnp.where(better, m, bv_ref[...])
    idx_ref[...] = jnp.where(better, jnp.broadcast_to(a + i * YB, (B, 16)),
                             idx_ref[...])
    nrm_ref[...] = jnp.where(better, jnp.broadcast_to(wr, (B, 16)),
                             nrm_ref[...])


def _tc_argmax(x2d, w_x2y, age):
    return pl.pallas_call(
        _argmax_body,
        grid=(NB,),
        in_specs=[
            pl.BlockSpec((B, X), lambda i: (0, 0)),
            pl.BlockSpec((YB, X), lambda i: (i, 0)),
            pl.BlockSpec((1, YB), lambda i: (0, i)),
        ],
        out_specs=[
            pl.BlockSpec((B, 16), lambda i: (0, 0)),
            pl.BlockSpec((B, 16), lambda i: (0, 0)),
        ],
        out_shape=[
            jax.ShapeDtypeStruct((B, 16), jnp.int32),
            jax.ShapeDtypeStruct((B, 16), jnp.float32),
        ],
        scratch_shapes=[pltpu.VMEM((B, 1), jnp.float32)],
    )(x2d, w_x2y, age)


_CHUNKS = ((0, 256), (256, 256), (512, 256), (768, 232))


def _sc_gather_body(idx_hbm, nrm_hbm, wx_hbm, wz_hbm, out_hbm, att_hbm,
                    bvec_v, nvec_v, blk0_v, blk1_v, col0_v, col1_v,
                    rblk0_v, rblk1_v, acol0_v, acol1_v,
                    sem_b0, sem_b1, sem_ai, sem_o, sem_ao):
    wid = lax.axis_index("s") * 2 + lax.axis_index("c")
    bs = [wid * 2, wid * 2 + 1]
    pltpu.sync_copy(idx_hbm.at[bs[0]], bvec_v.at[0])
    pltpu.sync_copy(idx_hbm.at[bs[1]], bvec_v.at[1])
    pltpu.sync_copy(nrm_hbm.at[bs[0]], nvec_v.at[0])
    pltpu.sync_copy(nrm_hbm.at[bs[1]], nvec_v.at[1])
    cs = [bvec_v[0][0], bvec_v[1][0]]
    blks, cols = [blk0_v, blk1_v], [col0_v, col1_v]
    rblks, acols = [rblk0_v, rblk1_v], [acol0_v, acol1_v]

    att_in = []
    for bl in range(2):
        r_al = pl.multiple_of((cs[bl] // 8) * 8, 8)
        cp = pltpu.make_async_copy(wx_hbm.at[pl.ds(r_al, 8), :],
                                   rblks[bl], sem_ai)
        cp.start()
        att_in.append(cp)

    jobs = [(bl, r0, n) for bl in range(2) for (r0, n) in _CHUNKS]
    sems = [sem_b0, sem_b1]

    def mk(j):
        bl, r0, n = jobs[j]
        c_al = pl.multiple_of((cs[bl] // 128) * 128, 128)
        return pltpu.make_async_copy(
            wz_hbm.at[pl.ds(r0, n), pl.ds(c_al, 128)],
            blks[j % 2].at[pl.ds(0, n), :], sems[j % 2])

    cops = [None] * len(jobs)
    for j in (0, 1):
        cops[j] = mk(j)
        cops[j].start()
    finals = []
    for j, (bl, r0, n) in enumerate(jobs):
        cops[j].wait()
        l = cs[bl] % 128
        blk, col = blks[j % 2], cols[bl]

        def peel(kc, _, r0=r0, l=l, blk=blk, col=col):
            col[pl.ds(r0 + kc, 16)] = blk[kc, pl.ds(l, 16)]
            return _

        lax.fori_loop(0, n, peel, None, unroll=8)
        if j + 2 < len(jobs):
            cops[j + 2] = mk(j + 2)
            cops[j + 2].start()
        if r0 + n >= Z:
            ocp = pltpu.make_async_copy(col, out_hbm.at[bs[bl]], sem_o)
            ocp.start()
            finals.append(ocp)
            att_in[bl].wait()
            jrow = cs[bl] % 8
            nrm = nvec_v[bl]
            for t in range(X // 16):
                acols[bl][pl.ds(t * 16, 16)] = (
                    rblks[bl][jrow, pl.ds(t * 16, 16)] / nrm)
            acp = pltpu.make_async_copy(acols[bl], att_hbm.at[bs[bl]], sem_ao)
            acp.start()
            finals.append(acp)
    for cp in finals:
        cp.wait()


@functools.cache
def _sc_gather():
    return pl.kernel(
        _sc_gather_body,
        mesh=plsc.VectorSubcoreMesh(core_axis_name="c", subcore_axis_name="s"),
        out_type=[
            jax.ShapeDtypeStruct((B, ZP), jnp.float32),
            jax.ShapeDtypeStruct((B, X), jnp.float32),
        ],
        scratch_types=[
            pltpu.VMEM((2, 16), jnp.int32),
            pltpu.VMEM((2, 16), jnp.float32),
            pltpu.VMEM((264, 128), jnp.float32),
            pltpu.VMEM((264, 128), jnp.float32),
            pltpu.VMEM((ZP,), jnp.float32),
            pltpu.VMEM((ZP,), jnp.float32),
            pltpu.VMEM((8, X), jnp.float32),
            pltpu.VMEM((8, X), jnp.float32),
            pltpu.VMEM((X,), jnp.float32),
            pltpu.VMEM((X,), jnp.float32),
            pltpu.SemaphoreType.DMA,
            pltpu.SemaphoreType.DMA,
            pltpu.SemaphoreType.DMA,
            pltpu.SemaphoreType.DMA,
            pltpu.SemaphoreType.DMA,
        ],
    )


def kernel(x, z, W_x2y, W_z2y, W_y2z, y_neuron_age, test_cnt):
    x2d = x.reshape(x.shape[0], -1)
    idx_b16, nrm_b16 = _tc_argmax(x2d, W_x2y, y_neuron_age)
    out_pad, att_weight = _sc_gather()(idx_b16, nrm_b16, W_x2y, W_y2z)
    return (out_pad[:, :Z], att_weight)


# speedup vs baseline: 1.2103x; 1.2103x over previous
import functools

import jax
import jax.numpy as jnp
from jax import lax
from jax.experimental import pallas as pl
from jax.experimental.pallas import tpu as pltpu
from jax.experimental.pallas import tpu_sc as plsc

B, Y, Z, X = 64, 32768, 1000, 256
YB = 4096
NB = Y // YB

ZP = 1024


def _argmax_body(x_ref, w_ref, age_ref, idx_ref, nrm_ref, bv_ref):
    i = pl.program_id(0)

    x = x_ref[...]
    xn = x / jnp.maximum(jnp.sqrt(jnp.sum(x * x, axis=1, keepdims=True)), 1e-12)

    w = w_ref[...]
    r = jnp.maximum(jnp.sqrt(jnp.sum(w * w, axis=1, keepdims=True)), 1e-12)
    wn = w / r

    y = lax.dot_general(xn, wn, (((1,), (1,)), ((), ())),
                        preferred_element_type=jnp.float32)
    age = age_ref[...]
    y = y * jnp.where(age >= 1.0, 1.0, 0.0)

    m = jnp.max(y, axis=1, keepdims=True)
    cols = lax.broadcasted_iota(jnp.int32, (B, YB), 1)
    a = jnp.min(jnp.where(y == m, cols, Y), axis=1, keepdims=True)

    onehot = (cols == a).astype(jnp.float32)
    wr = lax.dot_general(onehot, r, (((1,), (0,)), ((), ())),
                         precision=lax.Precision.HIGHEST,
                         preferred_element_type=jnp.float32)

    @pl.when(i == 0)
    def _():
        bv_ref[...] = jnp.full((B, 1), -jnp.inf, jnp.float32)

    better = m > bv_ref[...]
    bv_ref[...] = jnp.where(better, m, bv_ref[...])
    idx_ref[...] = jnp.where(better, jnp.broadcast_to(a + i * YB, (B, 16)),
                             idx_ref[...])
    nrm_ref[...] = jnp.where(better, jnp.broadcast_to(wr, (B, 16)),
                             nrm_ref[...])


def _tc_argmax(x2d, w_x2y, age):
    return pl.pallas_call(
        _argmax_body,
        grid=(NB,),
        in_specs=[
            pl.BlockSpec((B, X), lambda i: (0, 0)),
            pl.BlockSpec((YB, X), lambda i: (i, 0)),
            pl.BlockSpec((1, YB), lambda i: (0, i)),
        ],
        out_specs=[
            pl.BlockSpec((B, 16), lambda i: (0, 0)),
            pl.BlockSpec((B, 16), lambda i: (0, 0)),
        ],
        out_shape=[
            jax.ShapeDtypeStruct((B, 16), jnp.int32),
            jax.ShapeDtypeStruct((B, 16), jnp.float32),
        ],
        scratch_shapes=[pltpu.VMEM((B, 1), jnp.float32)],
    )(x2d, w_x2y, age)


_CHUNKS = ((0, 256), (256, 256), (512, 256), (768, 232))


def _sc_gather_body(idx_hbm, nrm_hbm, wx_hbm, wz_hbm, out_hbm, att_hbm,
                    bvec_v, nvec_v, blk0_v, blk1_v, col0_v, col1_v,
                    rblk0_v, rblk1_v, acol0_v, acol1_v,
                    sem_b0, sem_b1, sem_ai, sem_o, sem_ao):
    wid = lax.axis_index("s") * 2 + lax.axis_index("c")
    bs = [wid * 2, wid * 2 + 1]
    pltpu.sync_copy(idx_hbm.at[bs[0]], bvec_v.at[0])
    pltpu.sync_copy(idx_hbm.at[bs[1]], bvec_v.at[1])
    pltpu.sync_copy(nrm_hbm.at[bs[0]], nvec_v.at[0])
    pltpu.sync_copy(nrm_hbm.at[bs[1]], nvec_v.at[1])
    cs = [bvec_v[0][0], bvec_v[1][0]]
    blks, cols = [blk0_v, blk1_v], [col0_v, col1_v]
    rblks, acols = [rblk0_v, rblk1_v], [acol0_v, acol1_v]

    att_in = []
    for bl in range(2):
        r_al = pl.multiple_of((cs[bl] // 8) * 8, 8)
        cp = pltpu.make_async_copy(wx_hbm.at[pl.ds(r_al, 8), :],
                                   rblks[bl], sem_ai)
        cp.start()
        att_in.append(cp)

    jobs = [(bl, r0, n) for bl in range(2) for (r0, n) in _CHUNKS]
    sems = [sem_b0, sem_b1]

    def mk(j):
        bl, r0, n = jobs[j]
        c_al = pl.multiple_of((cs[bl] // 128) * 128, 128)
        return pltpu.make_async_copy(
            wz_hbm.at[pl.ds(r0, n), pl.ds(c_al, 128)],
            blks[j % 2].at[pl.ds(0, n), :], sems[j % 2])

    cops = [None] * len(jobs)
    for j in (0, 1):
        cops[j] = mk(j)
        cops[j].start()
    finals = []
    for j, (bl, r0, n) in enumerate(jobs):
        cops[j].wait()
        l = cs[bl] % 128
        blk, col = blks[j % 2], cols[bl]

        def peel(kc, _, r0=r0, l=l, blk=blk, col=col):
            col[pl.ds(r0 + kc, 16)] = blk[kc, pl.ds(l, 16)]
            return _

        lax.fori_loop(0, n, peel, None, unroll=8)
        if j + 2 < len(jobs):
            cops[j + 2] = mk(j + 2)
            cops[j + 2].start()
        if r0 + n >= Z:
            ocp = pltpu.make_async_copy(col, out_hbm.at[bs[bl]], sem_o)
            ocp.start()
            finals.append(ocp)
            att_in[bl].wait()
            jrow = cs[bl] % 8
            nrm = nvec_v[bl]
            for t in range(X // 16):
                acols[bl][pl.ds(t * 16, 16)] = (
                    rblks[bl][jrow, pl.ds(t * 16, 16)] / nrm)
            acp = pltpu.make_async_copy(acols[bl], att_hbm.at[bs[bl]], sem_ao)
            acp.start()
            finals.append(acp)
    for cp in finals:
        cp.wait()


@functools.cache
def _sc_gather():
    return pl.kernel(
        _sc_gather_body,
        mesh=plsc.VectorSubcoreMesh(core_axis_name="c", subcore_axis_name="s"),
        out_type=[
            jax.ShapeDtypeStruct((B, ZP), jnp.float32),
            jax.ShapeDtypeStruct((B, X), jnp.float32),
        ],
        scratch_types=[
            pltpu.VMEM((2, 16), jnp.int32),
            pltpu.VMEM((2, 16), jnp.float32),
            pltpu.VMEM((264, 128), jnp.float32),
            pltpu.VMEM((264, 128), jnp.float32),
            pltpu.VMEM((ZP,), jnp.float32),
            pltpu.VMEM((ZP,), jnp.float32),
            pltpu.VMEM((8, X), jnp.float32),
            pltpu.VMEM((8, X), jnp.float32),
            pltpu.VMEM((X,), jnp.float32),
            pltpu.VMEM((X,), jnp.float32),
            pltpu.SemaphoreType.DMA,
            pltpu.SemaphoreType.DMA,
            pltpu.SemaphoreType.DMA,
            pltpu.SemaphoreType.DMA,
            pltpu.SemaphoreType.DMA,
        ],
    )


def kernel(x, z, W_x2y, W_z2y, W_y2z, y_neuron_age, test_cnt):
    x2d = x.reshape(x.shape[0], -1)
    idx_b16, nrm_b16 = _tc_argmax(x2d, W_x2y, y_neuron_age)
    out_pad, att_weight = _sc_gather()(idx_b16, nrm_b16, W_x2y, W_y2z)
    return (out_pad[:, :Z], att_weight)
